# Initial kernel scaffold; baseline (speedup 1.0000x reference)
import jax
import jax.numpy as jnp
from jax.experimental import pallas as pl


def kernel(x, edge_index, batch, params):
    raise NotImplementedError("write your pallas kernel here")


# speedup vs baseline: 5.1600x; 5.1600x over previous
import functools

import jax
import jax.numpy as jnp
from jax import lax
from jax.experimental import pallas as pl
from jax.experimental.pallas import tpu as pltpu
from jax.experimental.pallas import tpu_sc as plsc

_N = 10000
_E = 160000
_D = 256
_Q = 64
_NQ = 4
_G = 64
_L = 4

_NTILES = 16
_EDGES_PER_TILE = _E // _NTILES
_CHUNK = 80
_NCHUNKS = _EDGES_PER_TILE // _CHUNK
_ROWS_PER_TILE = 624
_TAIL_ROWS = _N - _NTILES * _ROWS_PER_TILE

_BLK = 1000
_NBLK = _N // _BLK


_NBUF = 5
_NOUTER = _NCHUNKS // _NBUF


def _sc_aggregate(hq, src4, dst):
    mesh = plsc.VectorSubcoreMesh(core_axis_name="c", subcore_axis_name="s")

    scratch = (
        [pltpu.VMEM((_CHUNK,), jnp.int32) for _ in range(_NBUF)]
        + [pltpu.VMEM((_CHUNK,), jnp.int32) for _ in range(_NBUF)]
        + [pltpu.VMEM((_CHUNK, _Q), jnp.float32) for _ in range(_NBUF)]
        + [pltpu.VMEM((_ROWS_PER_TILE, _Q), jnp.float32)]
        + [pltpu.VMEM_SHARED((_N, _Q), jnp.float32)]
        + [pltpu.SemaphoreType.DMA for _ in range(3 * _NBUF)]
    )

    @functools.partial(
        pl.kernel,
        out_type=jax.ShapeDtypeStruct((_NQ * _N, _Q), jnp.float32),
        mesh=mesh,
        scratch_types=scratch,
        compiler_params=pltpu.CompilerParams(use_tc_tiling_on_sc=False),
    )
    def body(h_hbm, src_hbm, dst_hbm, out_hbm, *scr):
        srcv = scr[0:_NBUF]
        dstv = scr[_NBUF:2 * _NBUF]
        rows = scr[2 * _NBUF:3 * _NBUF]
        zbuf = scr[3 * _NBUF]
        acc = scr[3 * _NBUF + 1]
        isem = scr[3 * _NBUF + 2:3 * _NBUF + 2 + _NBUF]
        gsem = scr[3 * _NBUF + 2 + _NBUF:3 * _NBUF + 2 + 2 * _NBUF]
        ssem = scr[3 * _NBUF + 2 + 2 * _NBUF:]

        c = lax.axis_index("c")
        s = lax.axis_index("s")
        r0 = s * _ROWS_PER_TILE
        tail0 = _NTILES * _ROWS_PER_TILE

        def zstore(t, _):
            zbuf[t // 4, pl.ds((t % 4) * 16, 16)] = jnp.zeros((16,), jnp.float32)
            return 0

        lax.fori_loop(0, _ROWS_PER_TILE * 4, zstore, 0)

        for p in range(2):
            q = 2 * p + c

            pltpu.sync_copy(zbuf, acc.at[pl.ds(r0, _ROWS_PER_TILE)])

            @pl.when(s == _NTILES - 1)
            def _():
                pltpu.sync_copy(zbuf.at[pl.ds(0, _TAIL_ROWS)],
                                acc.at[pl.ds(tail0, _TAIL_ROWS)])

            plsc.subcore_barrier()

            def idx_start(j, b):
                e0 = s * _EDGES_PER_TILE + j * _CHUNK
                d1 = pltpu.async_copy(
                    src_hbm.at[pl.ds(q * _E + e0, _CHUNK)], srcv[b], isem[b])
                d2 = pltpu.async_copy(
                    dst_hbm.at[pl.ds(e0, _CHUNK)], dstv[b], isem[b])
                return d1, d2

            for b in range(_NBUF):
                idx_start(b, b)

            def outer(g2, _):
                for b in range(_NBUF):
                    @pl.when(g2 > 0)
                    def _(b=b):
                        pltpu.make_async_copy(
                            rows[b], acc.at[dstv[b]], ssem[b]).wait()
                        idx_start(g2 * _NBUF + b, b)
                gd = []
                for b in range(_NBUF):
                    pltpu.make_async_copy(
                        src_hbm.at[pl.ds(0, _CHUNK)], srcv[b], isem[b]).wait()
                    pltpu.make_async_copy(
                        dst_hbm.at[pl.ds(0, _CHUNK)], dstv[b], isem[b]).wait()
                    gd.append(pltpu.async_copy(h_hbm.at[srcv[b]], rows[b],
                                               gsem[b]))
                for b in range(_NBUF):
                    gd[b].wait()
                    pltpu.async_copy(rows[b], acc.at[dstv[b]], ssem[b],
                                     add=True)
                return 0

            lax.fori_loop(0, _NOUTER, outer, 0)
            for b in range(_NBUF):
                pltpu.make_async_copy(rows[b], acc.at[dstv[b]], ssem[b]).wait()
            plsc.subcore_barrier()

            pltpu.sync_copy(
                acc.at[pl.ds(r0, _ROWS_PER_TILE)],
                out_hbm.at[pl.ds(q * _N + r0, _ROWS_PER_TILE)],
            )

            @pl.when(s == _NTILES - 1)
            def _():
                pltpu.sync_copy(acc.at[pl.ds(tail0, _TAIL_ROWS)],
                                out_hbm.at[pl.ds(q * _N + tail0, _TAIL_ROWS)])

    return body(hq, src4, dst)


def _m1_body(h4_ref, a4_ref, w1_ref, b1_ref, w2_ref, b2_ref, ep_ref,
             t_ref, sum_ref, sq_ref):
    i = pl.program_id(0)
    h = jnp.concatenate([h4_ref[k] for k in range(_NQ)], axis=1)
    a = jnp.concatenate([a4_ref[k] for k in range(_NQ)], axis=1)
    z = h * ep_ref[0, 0] + a
    z = jnp.maximum(
        jnp.dot(z, w1_ref[...], preferred_element_type=jnp.float32) + b1_ref[...],
        0.0)
    z = jnp.maximum(
        jnp.dot(z, w2_ref[...], preferred_element_type=jnp.float32) + b2_ref[...],
        0.0)
    t_ref[...] = z
    ps = jnp.broadcast_to(jnp.sum(z, axis=0, keepdims=True), (8, _D))
    pq = jnp.broadcast_to(jnp.sum(z * z, axis=0, keepdims=True), (8, _D))

    @pl.when(i == 0)
    def _():
        sum_ref[...] = jnp.zeros_like(sum_ref)
        sq_ref[...] = jnp.zeros_like(sq_ref)

    sum_ref[...] += ps
    sq_ref[...] += pq


def _m1(h4, a4, w1, b1, w2, b2, onep):
    return pl.pallas_call(
        _m1_body,
        grid=(_NBLK,),
        in_specs=[
            pl.BlockSpec((_NQ, _BLK, _Q), lambda i: (0, i, 0)),
            pl.BlockSpec((_NQ, _BLK, _Q), lambda i: (0, i, 0)),
            pl.BlockSpec((_D, _D), lambda i: (0, 0)),
            pl.BlockSpec((1, _D), lambda i: (0, 0)),
            pl.BlockSpec((_D, _D), lambda i: (0, 0)),
            pl.BlockSpec((1, _D), lambda i: (0, 0)),
            pl.BlockSpec((1, 1), lambda i: (0, 0)),
        ],
        out_specs=[
            pl.BlockSpec((_BLK, _D), lambda i: (i, 0)),
            pl.BlockSpec((8, _D), lambda i: (0, 0)),
            pl.BlockSpec((8, _D), lambda i: (0, 0)),
        ],
        out_shape=[
            jax.ShapeDtypeStruct((_N, _D), jnp.float32),
            jax.ShapeDtypeStruct((8, _D), jnp.float32),
            jax.ShapeDtypeStruct((8, _D), jnp.float32),
        ],
        compiler_params=pltpu.CompilerParams(
            dimension_semantics=("arbitrary",)),
    )(h4, a4, w1, b1, w2, b2, onep)


def _m2_body(t_ref, sum_ref, sq_ref, g_ref, be_ref, batch_ref,
             h4_ref, pool_ref, cnt_ref):
    i = pl.program_id(0)
    inv_n = 1.0 / _N
    mean = sum_ref[0:1, :] * inv_n
    var = sq_ref[0:1, :] * inv_n - mean * mean
    scale = lax.rsqrt(var + 1e-5) * g_ref[...]
    z = (t_ref[...] - mean) * scale + be_ref[...]
    for k in range(_NQ):
        h4_ref[k] = z[:, k * _Q:(k + 1) * _Q]
    b = batch_ref[0, 0]
    oh = (b[:, None] == lax.broadcasted_iota(jnp.int32, (_BLK, _G), 1)
          ).astype(jnp.float32)
    pp = lax.dot_general(oh, z, (((0,), (0,)), ((), ())),
                         preferred_element_type=jnp.float32)
    pc = lax.dot_general(oh, jnp.ones((_BLK, 128), jnp.float32),
                         (((0,), (0,)), ((), ())),
                         preferred_element_type=jnp.float32)

    @pl.when(i == 0)
    def _():
        pool_ref[...] = jnp.zeros_like(pool_ref)
        cnt_ref[...] = jnp.zeros_like(cnt_ref)

    pool_ref[...] += pp
    cnt_ref[...] += pc


def _m2(t, ssum, sq, gamma, beta, batch_r):
    return pl.pallas_call(
        _m2_body,
        grid=(_NBLK,),
        in_specs=[
            pl.BlockSpec((_BLK, _D), lambda i: (i, 0)),
            pl.BlockSpec((8, _D), lambda i: (0, 0)),
            pl.BlockSpec((8, _D), lambda i: (0, 0)),
            pl.BlockSpec((1, _D), lambda i: (0, 0)),
            pl.BlockSpec((1, _D), lambda i: (0, 0)),
            pl.BlockSpec((1, 1, _BLK), lambda i: (i, 0, 0)),
        ],
        out_specs=[
            pl.BlockSpec((_NQ, _BLK, _Q), lambda i: (0, i, 0)),
            pl.BlockSpec((_G, _D), lambda i: (0, 0)),
            pl.BlockSpec((_G, 128), lambda i: (0, 0)),
        ],
        out_shape=[
            jax.ShapeDtypeStruct((_NQ, _N, _Q), jnp.float32),
            jax.ShapeDtypeStruct((_G, _D), jnp.float32),
            jax.ShapeDtypeStruct((_G, 128), jnp.float32),
        ],
        compiler_params=pltpu.CompilerParams(
            dimension_semantics=("arbitrary",)),
    )(t, ssum, sq, gamma, beta, batch_r)


def _head_body(p0, p1, p2, p3, cnt, w1, b1, w2, b2, w3, b3, w4, b4, out):
    ge = jnp.concatenate([p0[...], p1[...], p2[...], p3[...]], axis=1)
    ge = ge / jnp.maximum(cnt[:, 0:1], 1.0)
    g = jnp.maximum(
        jnp.dot(ge, w1[...], preferred_element_type=jnp.float32) + b1[...], 0.0)
    g = jnp.maximum(
        jnp.dot(g, w2[...], preferred_element_type=jnp.float32) + b2[...], 0.0)
    g = jnp.maximum(
        jnp.dot(g, w3[...], preferred_element_type=jnp.float32) + b3[...], 0.0)
    lg = jnp.dot(g, w4[...], preferred_element_type=jnp.float32) + b4[...]
    m = jnp.max(lg, axis=1, keepdims=True)
    e = jnp.exp(lg - m)
    out[...] = lg - m - jnp.log(jnp.sum(e, axis=1, keepdims=True))


def _head(pools, cnt, c):
    nc = c['W4'].shape[1]
    w4p = jnp.pad(c['W4'], ((0, 0), (0, 128 - nc)))
    b4p = jnp.pad(c['b4'].reshape(1, -1), ((0, 0), (0, 128 - nc)),
                  constant_values=-1e30)
    out = pl.pallas_call(
        _head_body,
        out_shape=jax.ShapeDtypeStruct((_G, 128), jnp.float32),
    )(pools[0], pools[1], pools[2], pools[3], cnt,
      c['W1'], c['b1'].reshape(1, -1),
      c['W2'], c['b2'].reshape(1, -1),
      c['W3'], c['b3'].reshape(1, -1),
      w4p, b4p)
    return out[:, :nc]


def kernel(x, edge_index, batch, params):
    src = edge_index[0]
    dst = edge_index[1]
    src4 = jnp.concatenate([src + q * _N for q in range(_NQ)])
    batch_r = batch.reshape(_NBLK, 1, _BLK)

    h4 = x.reshape(_N, _NQ, _Q).transpose(1, 0, 2)
    hq = h4.reshape(_NQ * _N, _Q)

    pools = []
    cnt = None
    for li in range(_L):
        p = params['gin%d' % li]
        aggr = _sc_aggregate(hq, src4, dst)
        a4 = aggr.reshape(_NQ, _N, _Q)
        onep = (1.0 + p['eps']).reshape(1, 1)
        t, ssum, sq = _m1(h4, a4, p['W1'], p['b1'].reshape(1, -1),
                          p['W2'], p['b2'].reshape(1, -1), onep)
        h4, pool_i, cnt_i = _m2(t, ssum, sq, p['gamma'].reshape(1, -1),
                                p['beta'].reshape(1, -1), batch_r)
        hq = h4.reshape(_NQ * _N, _Q)
        pools.append(pool_i)
        if cnt is None:
            cnt = cnt_i

    return _head(pools, cnt, params['cls'])


# speedup vs baseline: 6.3231x; 1.2254x over previous
import functools

import jax
import jax.numpy as jnp
from jax import lax
from jax.experimental import pallas as pl
from jax.experimental.pallas import tpu as pltpu
from jax.experimental.pallas import tpu_sc as plsc

_N = 10000
_E = 160000
_D = 256
_Q = 64
_NQ = 4
_G = 64
_L = 4

_NTILES = 16
_EDGES_PER_TILE = _E // _NTILES
_CHUNK = 80
_NCHUNKS = _EDGES_PER_TILE // _CHUNK
_ROWS_PER_TILE = 624
_TAIL_ROWS = _N - _NTILES * _ROWS_PER_TILE

_BLK = 1000
_NBLK = _N // _BLK


_NBUF = 5
_NOUTER = _NCHUNKS // _NBUF
_VPC = _CHUNK // 16


def _sc_aggregate(hq, src_t, dst_t):
    mesh = plsc.VectorSubcoreMesh(core_axis_name="c", subcore_axis_name="s")

    scratch = (
        [pltpu.VMEM((_CHUNK,), jnp.int32) for _ in range(_NBUF)]
        + [pltpu.VMEM((_CHUNK, _Q), jnp.float32) for _ in range(_NBUF)]
        + [
            pltpu.VMEM((_NCHUNKS, _CHUNK), jnp.int32),
            pltpu.VMEM((_NCHUNKS, _CHUNK), jnp.int32),
            pltpu.VMEM((_ROWS_PER_TILE, _Q), jnp.float32),
            pltpu.VMEM_SHARED((_N, _Q), jnp.float32),
        ]
        + [pltpu.SemaphoreType.DMA for _ in range(2 * _NBUF)]
    )

    @functools.partial(
        pl.kernel,
        out_type=jax.ShapeDtypeStruct((_NQ * _N, _Q), jnp.float32),
        mesh=mesh,
        scratch_types=scratch,
        compiler_params=pltpu.CompilerParams(use_tc_tiling_on_sc=False),
    )
    def body(h_hbm, src_hbm, dst_hbm, out_hbm, *scr):
        srcq = scr[0:_NBUF]
        rows = scr[_NBUF:2 * _NBUF]
        src_all = scr[2 * _NBUF]
        dst_all = scr[2 * _NBUF + 1]
        zbuf = scr[2 * _NBUF + 2]
        acc = scr[2 * _NBUF + 3]
        gsem = scr[2 * _NBUF + 4:2 * _NBUF + 4 + _NBUF]
        ssem = scr[2 * _NBUF + 4 + _NBUF:]

        c = lax.axis_index("c")
        s = lax.axis_index("s")
        r0 = s * _ROWS_PER_TILE
        tail0 = _NTILES * _ROWS_PER_TILE

        pltpu.async_copy(src_hbm.at[s], src_all, gsem[0])
        pltpu.async_copy(dst_hbm.at[s], dst_all, gsem[0])

        def zstore(t, _):
            zbuf[t // 4, pl.ds((t % 4) * 16, 16)] = jnp.zeros((16,), jnp.float32)
            return 0

        lax.fori_loop(0, _ROWS_PER_TILE * 4, zstore, 0)
        pltpu.make_async_copy(src_hbm.at[s], src_all, gsem[0]).wait()
        pltpu.make_async_copy(dst_hbm.at[s], dst_all, gsem[0]).wait()

        for p in range(2):
            q = 2 * p + c

            pltpu.sync_copy(zbuf, acc.at[pl.ds(r0, _ROWS_PER_TILE)])

            @pl.when(s == _NTILES - 1)
            def _():
                pltpu.sync_copy(zbuf.at[pl.ds(0, _TAIL_ROWS)],
                                acc.at[pl.ds(tail0, _TAIL_ROWS)])

            plsc.subcore_barrier()

            def outer(g2, _):
                gd = []
                for b in range(_NBUF):
                    j = g2 * _NBUF + b

                    @pl.when(g2 > 0)
                    def _(b=b, j=j):
                        pltpu.make_async_copy(
                            rows[b], acc.at[dst_all.at[j]], ssem[b]).wait()
                    for k in range(_VPC):
                        srcq[b][pl.ds(k * 16, 16)] = (
                            src_all[j, pl.ds(k * 16, 16)] + q)
                    gd.append(pltpu.async_copy(h_hbm.at[srcq[b]], rows[b],
                                               gsem[b]))
                for b in range(_NBUF):
                    j = g2 * _NBUF + b
                    gd[b].wait()
                    pltpu.async_copy(rows[b], acc.at[dst_all.at[j]], ssem[b],
                                     add=True)
                return 0

            lax.fori_loop(0, _NOUTER, outer, 0)
            for b in range(_NBUF):
                pltpu.make_async_copy(
                    rows[b], acc.at[dst_all.at[_NCHUNKS - _NBUF + b]],
                    ssem[b]).wait()
            plsc.subcore_barrier()

            pltpu.sync_copy(
                acc.at[pl.ds(r0, _ROWS_PER_TILE)],
                out_hbm.at[pl.ds(q * _N + r0, _ROWS_PER_TILE)],
            )

            @pl.when(s == _NTILES - 1)
            def _():
                pltpu.sync_copy(acc.at[pl.ds(tail0, _TAIL_ROWS)],
                                out_hbm.at[pl.ds(q * _N + tail0, _TAIL_ROWS)])

    return body(hq, src_t, dst_t)


def _m1_body(h_ref, a4_ref, w1_ref, b1_ref, w2_ref, b2_ref, ep_ref,
             t_ref, sum_ref, sq_ref):
    i = pl.program_id(0)
    h = h_ref[...]
    a = jnp.concatenate([a4_ref[k] for k in range(_NQ)], axis=1)
    z = h * ep_ref[0, 0] + a
    z = jnp.maximum(
        jnp.dot(z, w1_ref[...], preferred_element_type=jnp.float32) + b1_ref[...],
        0.0)
    z = jnp.maximum(
        jnp.dot(z, w2_ref[...], preferred_element_type=jnp.float32) + b2_ref[...],
        0.0)
    t_ref[...] = z
    ps = jnp.broadcast_to(jnp.sum(z, axis=0, keepdims=True), (8, _D))
    pq = jnp.broadcast_to(jnp.sum(z * z, axis=0, keepdims=True), (8, _D))

    @pl.when(i == 0)
    def _():
        sum_ref[...] = jnp.zeros_like(sum_ref)
        sq_ref[...] = jnp.zeros_like(sq_ref)

    sum_ref[...] += ps
    sq_ref[...] += pq


def _m1(h, a4, w1, b1, w2, b2, onep):
    return pl.pallas_call(
        _m1_body,
        grid=(_NBLK,),
        in_specs=[
            pl.BlockSpec((_BLK, _D), lambda i: (i, 0)),
            pl.BlockSpec((_NQ, _BLK, _Q), lambda i: (0, i, 0)),
            pl.BlockSpec((_D, _D), lambda i: (0, 0)),
            pl.BlockSpec((1, _D), lambda i: (0, 0)),
            pl.BlockSpec((_D, _D), lambda i: (0, 0)),
            pl.BlockSpec((1, _D), lambda i: (0, 0)),
            pl.BlockSpec((1, 1), lambda i: (0, 0)),
        ],
        out_specs=[
            pl.BlockSpec((_BLK, _D), lambda i: (i, 0)),
            pl.BlockSpec((8, _D), lambda i: (0, 0)),
            pl.BlockSpec((8, _D), lambda i: (0, 0)),
        ],
        out_shape=[
            jax.ShapeDtypeStruct((_N, _D), jnp.float32),
            jax.ShapeDtypeStruct((8, _D), jnp.float32),
            jax.ShapeDtypeStruct((8, _D), jnp.float32),
        ],
        compiler_params=pltpu.CompilerParams(
            dimension_semantics=("arbitrary",)),
    )(h, a4, w1, b1, w2, b2, onep)


def _m2_body(t_ref, sum_ref, sq_ref, g_ref, be_ref, batch_ref,
             h_ref, pool_ref, cnt_ref):
    i = pl.program_id(0)
    inv_n = 1.0 / _N
    mean = sum_ref[0:1, :] * inv_n
    var = sq_ref[0:1, :] * inv_n - mean * mean
    scale = lax.rsqrt(var + 1e-5) * g_ref[...]
    z = (t_ref[...] - mean) * scale + be_ref[...]
    h_ref[...] = z
    b = batch_ref[0, 0]
    oh = (b[:, None] == lax.broadcasted_iota(jnp.int32, (_BLK, _G), 1)
          ).astype(jnp.float32)
    pp = lax.dot_general(oh, z, (((0,), (0,)), ((), ())),
                         preferred_element_type=jnp.float32)
    pc = lax.dot_general(oh, jnp.ones((_BLK, 128), jnp.float32),
                         (((0,), (0,)), ((), ())),
                         preferred_element_type=jnp.float32)

    @pl.when(i == 0)
    def _():
        pool_ref[...] = jnp.zeros_like(pool_ref)
        cnt_ref[...] = jnp.zeros_like(cnt_ref)

    pool_ref[...] += pp
    cnt_ref[...] += pc


def _m2(t, ssum, sq, gamma, beta, batch_r):
    return pl.pallas_call(
        _m2_body,
        grid=(_NBLK,),
        in_specs=[
            pl.BlockSpec((_BLK, _D), lambda i: (i, 0)),
            pl.BlockSpec((8, _D), lambda i: (0, 0)),
            pl.BlockSpec((8, _D), lambda i: (0, 0)),
            pl.BlockSpec((1, _D), lambda i: (0, 0)),
            pl.BlockSpec((1, _D), lambda i: (0, 0)),
            pl.BlockSpec((1, 1, _BLK), lambda i: (i, 0, 0)),
        ],
        out_specs=[
            pl.BlockSpec((_BLK, _D), lambda i: (i, 0)),
            pl.BlockSpec((_G, _D), lambda i: (0, 0)),
            pl.BlockSpec((_G, 128), lambda i: (0, 0)),
        ],
        out_shape=[
            jax.ShapeDtypeStruct((_N, _D), jnp.float32),
            jax.ShapeDtypeStruct((_G, _D), jnp.float32),
            jax.ShapeDtypeStruct((_G, 128), jnp.float32),
        ],
        compiler_params=pltpu.CompilerParams(
            dimension_semantics=("arbitrary",)),
    )(t, ssum, sq, gamma, beta, batch_r)


def _head_body(p0, p1, p2, p3, cnt, w1, b1, w2, b2, w3, b3, w4, b4, out):
    ge = jnp.concatenate([p0[...], p1[...], p2[...], p3[...]], axis=1)
    ge = ge / jnp.maximum(cnt[:, 0:1], 1.0)
    g = jnp.maximum(
        jnp.dot(ge, w1[...], preferred_element_type=jnp.float32) + b1[...], 0.0)
    g = jnp.maximum(
        jnp.dot(g, w2[...], preferred_element_type=jnp.float32) + b2[...], 0.0)
    g = jnp.maximum(
        jnp.dot(g, w3[...], preferred_element_type=jnp.float32) + b3[...], 0.0)
    lg = jnp.dot(g, w4[...], preferred_element_type=jnp.float32) + b4[...]
    m = jnp.max(lg, axis=1, keepdims=True)
    e = jnp.exp(lg - m)
    out[...] = lg - m - jnp.log(jnp.sum(e, axis=1, keepdims=True))


def _head(pools, cnt, c):
    nc = c['W4'].shape[1]
    w4p = jnp.pad(c['W4'], ((0, 0), (0, 128 - nc)))
    b4p = jnp.pad(c['b4'].reshape(1, -1), ((0, 0), (0, 128 - nc)),
                  constant_values=-1e30)
    out = pl.pallas_call(
        _head_body,
        out_shape=jax.ShapeDtypeStruct((_G, 128), jnp.float32),
    )(pools[0], pools[1], pools[2], pools[3], cnt,
      c['W1'], c['b1'].reshape(1, -1),
      c['W2'], c['b2'].reshape(1, -1),
      c['W3'], c['b3'].reshape(1, -1),
      w4p, b4p)
    return out[:, :nc]


def kernel(x, edge_index, batch, params):
    src = edge_index[0]
    dst = edge_index[1]
    src_t = (src * _NQ).reshape(_NTILES, _NCHUNKS, _CHUNK)
    dst_t = dst.reshape(_NTILES, _NCHUNKS, _CHUNK)
    batch_r = batch.reshape(_NBLK, 1, _BLK)

    h = x
    pools = []
    cnt = None
    for li in range(_L):
        p = params['gin%d' % li]
        aggr = _sc_aggregate(h.reshape(_NQ * _N, _Q), src_t, dst_t)
        a4 = aggr.reshape(_NQ, _N, _Q)
        onep = (1.0 + p['eps']).reshape(1, 1)
        t, ssum, sq = _m1(h, a4, p['W1'], p['b1'].reshape(1, -1),
                          p['W2'], p['b2'].reshape(1, -1), onep)
        h, pool_i, cnt_i = _m2(t, ssum, sq, p['gamma'].reshape(1, -1),
                               p['beta'].reshape(1, -1), batch_r)
        pools.append(pool_i)
        if cnt is None:
            cnt = cnt_i

    return _head(pools, cnt, params['cls'])
